# baseline (speedup 1.0000x reference)
import functools

import jax
import jax.numpy as jnp
from jax import lax
from jax.experimental import pallas as pl
from jax.experimental.pallas import tpu as pltpu
from jax.experimental.pallas import tpu_sc as plsc

N = 100000
E = 1600000
H = 32
OUT = 16
NMID = 10

K = 6
EP = 33 * (128 * 32 * K * 2)
RP = EP // 128
RT_MID = RP // 16
RT_SPLIT = RP // 32
CH = 6272
EPS = 1e-5

RB = 2000
GRID = N // RB


def _make_sc_conv(two_tables: bool):
    nblocks = RT_MID // K if two_tables else RT_SPLIT // K
    npairs = nblocks // 2
    mesh = plsc.VectorSubcoreMesh(core_axis_name="c", subcore_axis_name="s")

    @functools.partial(
        pl.kernel,
        out_type=jax.ShapeDtypeStruct((2, N, 16), jnp.float32),
        mesh=mesh,
        scratch_types=[
            pltpu.VMEM((K, 2, 128), jnp.int32),
            pltpu.VMEM((K, 2, 128), jnp.int32),
            pltpu.VMEM((K, 128), jnp.float32),
            pltpu.VMEM((K, 128), jnp.float32),
            pltpu.VMEM((K, 128), jnp.int32),
            pltpu.VMEM((K, 128), jnp.int32),
            pltpu.VMEM((K * 128, 16), jnp.float32),
            pltpu.VMEM((K * 128, 16), jnp.float32),
            pltpu.VMEM_SHARED((N, 16), jnp.float32),
            pltpu.SemaphoreType.DMA,
            pltpu.SemaphoreType.DMA,
            pltpu.SemaphoreType.DMA,
            pltpu.SemaphoreType.DMA,
            pltpu.SemaphoreType.DMA,
            pltpu.SemaphoreType.DMA,
        ],
        compiler_params=pltpu.CompilerParams(use_tc_tiling_on_sc=False),
    )
    def conv(table, edges, wvals, zeros, out, idx0, idx1, w0, w1, dst0, dst1,
             rows0, rows1, acc, sem_i0, sem_i1, sem_g0, sem_g1, sem_s0,
             sem_s1):
        c = lax.axis_index("c")
        s = lax.axis_index("s")
        idx_b = (idx0, idx1)
        w_b = (w0, w1)
        dst_b = (dst0, dst1)
        rows_b = (rows0, rows1)
        sem_i = (sem_i0, sem_i1)
        sem_g = (sem_g0, sem_g1)
        sem_s = (sem_s0, sem_s1)
        n0 = jnp.minimum(s * CH, N - CH)
        pltpu.sync_copy(zeros.at[pl.ds(n0, CH)], acc.at[pl.ds(n0, CH)])
        plsc.subcore_barrier()
        if two_tables:
            row0 = s * RT_MID
            coff = c * N
        else:
            row0 = (c * 16 + s) * RT_SPLIT
            coff = None

        def fetch_idx(b, p):
            r = jnp.minimum(row0 + b * K, RP - K)
            d1 = pltpu.async_copy(edges.at[pl.ds(r, K)], idx_b[p], sem_i[p])
            d2 = pltpu.async_copy(wvals.at[pl.ds(r, K)], w_b[p], sem_i[p])
            return (d1, d2)

        def add_coff(p):
            if coff is None:
                return
            iv = idx_b[p]
            def body(q, car):
                sl = pl.ds((q % 8) * 16, 16)
                iv[q // 8, 0, sl] = iv[q // 8, 0, sl] + coff
                return car
            lax.fori_loop(0, K * 8, body, 0, unroll=2)

        def fire_gathers(p):
            return [
                pltpu.async_copy(table.at[idx_b[p].at[j, 0]],
                                 rows_b[p].at[pl.ds(j * 128, 128)], sem_g[p])
                for j in range(K)
            ]

        def mul_and_dst(p):
            iv, wv, dv, rv = idx_b[p], w_b[p], dst_b[p], rows_b[p]
            def body(q, car):
                j = q // 8
                sl = pl.ds((q % 8) * 16, 16)
                wc = wv[j, sl]
                dv[j, sl] = iv[j, 1, sl]
                base = q * 16
                for l in range(16):
                    rv[base + l, :] = rv[base + l, :] * wc[l]
                return car
            lax.fori_loop(0, K * 8, body, 0, unroll=4)

        def fire_scatters(p):
            return [
                pltpu.async_copy(rows_b[p].at[pl.ds(j * 128, 128)],
                                 acc.at[dst_b[p].at[j]], sem_s[p], add=True)
                for j in range(K)
            ]

        for d in fetch_idx(0, 0):
            d.wait()
        for d in fetch_idx(1, 1):
            d.wait()
        add_coff(0)
        add_coff(1)
        fire_gathers(0)
        fire_gathers(1)

        def pair_body(t, carry):
            for p in range(2):
                for j in range(K):
                    pltpu.make_async_copy(
                        table.at[idx_b[p].at[j, 0]],
                        rows_b[p].at[pl.ds(j * 128, 128)], sem_g[p]).wait()
                mul_and_dst(p)
                fire_scatters(p)
                dis = fetch_idx(2 * t + 2 + p, p)
                for j in range(K):
                    pltpu.make_async_copy(
                        rows_b[p].at[pl.ds(j * 128, 128)],
                        acc.at[dst_b[p].at[j]], sem_s[p]).wait()
                for d in dis:
                    d.wait()
                add_coff(p)
                fire_gathers(p)
            return carry

        lax.fori_loop(0, npairs, pair_body, 0)
        for p in range(2):
            for j in range(K):
                pltpu.make_async_copy(
                    table.at[idx_b[p].at[j, 0]],
                    rows_b[p].at[pl.ds(j * 128, 128)], sem_g[p]).wait()
        plsc.subcore_barrier()
        pltpu.sync_copy(acc.at[pl.ds(n0, CH)],
                        out.at[c, pl.ds(n0, CH)])

    return conv


_sc_mid = _make_sc_conv(True)
_sc_split = _make_sc_conv(False)


def _stats2(v):
    mu_b = jnp.mean(v, axis=0)
    d = v - mu_b
    return jnp.stack([mu_b, jnp.sum(d * d, axis=0)])[None]


def _combine(bs):
    mu = jnp.mean(bs[:, 0, :], axis=0)
    dmu = bs[:, 0, :] - mu
    var = (jnp.sum(bs[:, 1, :], axis=0) + RB * jnp.sum(dmu * dmu, axis=0)) / N
    return mu, lax.rsqrt(var + EPS)


def _bn0_stats_body(x_ref, st_ref):
    st_ref[...] = _stats2(x_ref[...])


def _zfirst_body(x_ref, st_ref, g_ref, b_ref, w_ref, bb_ref, z_ref):
    mu, rstd = _combine(st_ref[...])
    xb = (x_ref[...] - mu) * rstd * g_ref[...] + b_ref[...]
    z = jnp.dot(xb, w_ref[...], preferred_element_type=jnp.float32)
    z = z + bb_ref[...]
    z_ref[0] = z[:, 0:16]
    z_ref[1] = z[:, 16:32]


def _zfirst(x, g, b, w, bb):
    st = pl.pallas_call(
        _bn0_stats_body,
        grid=(GRID,),
        in_specs=[pl.BlockSpec((RB, 2), lambda i: (i, 0))],
        out_specs=pl.BlockSpec((1, 2, 2), lambda i: (i, 0, 0)),
        out_shape=jax.ShapeDtypeStruct((GRID, 2, 2), jnp.float32),
    )(x)
    return pl.pallas_call(
        _zfirst_body,
        grid=(GRID,),
        in_specs=[
            pl.BlockSpec((RB, 2), lambda i: (i, 0)),
            pl.BlockSpec((GRID, 2, 2), lambda i: (0, 0, 0)),
            pl.BlockSpec((1, 2), lambda i: (0, 0)),
            pl.BlockSpec((1, 2), lambda i: (0, 0)),
            pl.BlockSpec((2, H), lambda i: (0, 0)),
            pl.BlockSpec((1, H), lambda i: (0, 0)),
        ],
        out_specs=pl.BlockSpec((2, RB, 16), lambda i: (0, i, 0)),
        out_shape=jax.ShapeDtypeStruct((2, N, 16), jnp.float32),
    )(x, st, g, b, w, bb)


def _mid_stats_body(s_ref, st_ref):
    sv = jnp.concatenate([s_ref[0], s_ref[1]], axis=1)
    st_ref[...] = _stats2(sv)


def _mid_stats(s):
    return pl.pallas_call(
        _mid_stats_body,
        grid=(GRID,),
        in_specs=[pl.BlockSpec((2, RB, 16), lambda i: (0, i, 0))],
        out_specs=pl.BlockSpec((1, 2, H), lambda i: (i, 0, 0)),
        out_shape=jax.ShapeDtypeStruct((GRID, 2, H), jnp.float32),
    )(s)


def _mid_z_body(s_ref, st_ref, g_ref, b_ref, w_ref, bb_ref, z_ref):
    mu, rstd = _combine(st_ref[...])
    sv = jnp.concatenate([s_ref[0], s_ref[1]], axis=1)
    h = (sv - mu) * rstd * g_ref[...] + b_ref[...]
    h = jnp.maximum(h, 0.0)
    z = jnp.dot(h, w_ref[...], preferred_element_type=jnp.float32)
    z = z + bb_ref[...]
    z_ref[0] = z[:, 0:16]
    z_ref[1] = z[:, 16:32]


def _mid_z(s, st, g, b, w, bb):
    return pl.pallas_call(
        _mid_z_body,
        grid=(GRID,),
        in_specs=[
            pl.BlockSpec((2, RB, 16), lambda i: (0, i, 0)),
            pl.BlockSpec((GRID, 2, H), lambda i: (0, 0, 0)),
            pl.BlockSpec((1, H), lambda i: (0, 0)),
            pl.BlockSpec((1, H), lambda i: (0, 0)),
            pl.BlockSpec((H, H), lambda i: (0, 0)),
            pl.BlockSpec((1, H), lambda i: (0, 0)),
        ],
        out_specs=pl.BlockSpec((2, RB, 16), lambda i: (0, i, 0)),
        out_shape=jax.ShapeDtypeStruct((2, N, 16), jnp.float32),
    )(s, st, g, b, w, bb)


def _zlast_body(s_ref, st_ref, g_ref, b_ref, w_ref, bb_ref, z_ref):
    mu, rstd = _combine(st_ref[...])
    sv = jnp.concatenate([s_ref[0], s_ref[1]], axis=1)
    h = (sv - mu) * rstd * g_ref[...] + b_ref[...]
    h = jnp.maximum(h, 0.0)
    z = jnp.dot(h, w_ref[...], preferred_element_type=jnp.float32)
    z_ref[...] = z + bb_ref[...]


def _zlast(s, st, g, b, w, bb):
    return pl.pallas_call(
        _zlast_body,
        grid=(GRID,),
        in_specs=[
            pl.BlockSpec((2, RB, 16), lambda i: (0, i, 0)),
            pl.BlockSpec((GRID, 2, H), lambda i: (0, 0, 0)),
            pl.BlockSpec((1, H), lambda i: (0, 0)),
            pl.BlockSpec((1, H), lambda i: (0, 0)),
            pl.BlockSpec((H, OUT), lambda i: (0, 0)),
            pl.BlockSpec((1, OUT), lambda i: (0, 0)),
        ],
        out_specs=pl.BlockSpec((RB, OUT), lambda i: (i, 0)),
        out_shape=jax.ShapeDtypeStruct((N, OUT), jnp.float32),
    )(s, st, g, b, w, bb)


def _fin_stats_body(s_ref, st_ref):
    st_ref[...] = _stats2(s_ref[0] + s_ref[1])


def _fin_apply_body(s_ref, st_ref, g_ref, b_ref, o_ref):
    mu, rstd = _combine(st_ref[...])
    o = s_ref[0] + s_ref[1]
    o_ref[...] = (o - mu) * rstd * g_ref[...] + b_ref[...]


def _final(s, g, b):
    st = pl.pallas_call(
        _fin_stats_body,
        grid=(GRID,),
        in_specs=[pl.BlockSpec((2, RB, 16), lambda i: (0, i, 0))],
        out_specs=pl.BlockSpec((1, 2, OUT), lambda i: (i, 0, 0)),
        out_shape=jax.ShapeDtypeStruct((GRID, 2, OUT), jnp.float32),
    )(s)
    return pl.pallas_call(
        _fin_apply_body,
        grid=(GRID,),
        in_specs=[
            pl.BlockSpec((2, RB, 16), lambda i: (0, i, 0)),
            pl.BlockSpec((GRID, 2, OUT), lambda i: (0, 0, 0)),
            pl.BlockSpec((1, OUT), lambda i: (0, 0)),
            pl.BlockSpec((1, OUT), lambda i: (0, 0)),
        ],
        out_specs=pl.BlockSpec((RB, OUT), lambda i: (i, 0)),
        out_shape=jax.ShapeDtypeStruct((N, OUT), jnp.float32),
    )(s, st, g, b)


def kernel(x, edge_index, edge_weight, bn0_gamma, bn0_beta, W_first, b_first,
           bn1_gamma, bn1_beta, W_mid, b_mid, bnm_gamma, bnm_beta,
           W_last, b_last, bnl_gamma, bnl_beta):
    pad = EP - E
    src = jnp.concatenate([edge_index[0], jnp.zeros((pad,), jnp.int32)])
    dst = jnp.concatenate([edge_index[1], jnp.zeros((pad,), jnp.int32)])
    wgt = jnp.concatenate([edge_weight, jnp.zeros((pad,), jnp.float32)])
    edges = jnp.stack([src.reshape(RP, 128), dst.reshape(RP, 128)], axis=1)
    wvals = wgt.reshape(RP, 128)
    zeros16 = jnp.zeros((N, 16), jnp.float32)

    z = _zfirst(x, bn0_gamma.reshape(1, 2), bn0_beta.reshape(1, 2),
                W_first, b_first.reshape(1, H))
    s = _sc_mid(z.reshape(2 * N, 16), edges, wvals, zeros16)
    st = _mid_stats(s)
    g, b = bn1_gamma, bn1_beta
    for i in range(NMID):
        z = _mid_z(s, st, g.reshape(1, H), b.reshape(1, H),
                   W_mid[i], b_mid[i].reshape(1, H))
        s = _sc_mid(z.reshape(2 * N, 16), edges, wvals, zeros16)
        st = _mid_stats(s)
        g, b = bnm_gamma[i], bnm_beta[i]
    zl = _zlast(s, st, g.reshape(1, H), b.reshape(1, H),
                W_last, b_last.reshape(1, OUT))
    sl = _sc_split(zl, edges, wvals, zeros16)
    return _final(sl, bnl_gamma.reshape(1, OUT), bnl_beta.reshape(1, OUT))


# speedup vs baseline: 1.3477x; 1.3477x over previous
import functools

import jax
import jax.numpy as jnp
from jax import lax
from jax.experimental import pallas as pl
from jax.experimental.pallas import tpu as pltpu
from jax.experimental.pallas import tpu_sc as plsc

N = 100000
E = 1600000
H = 32
OUT = 16
NMID = 10

K = 6
EP = 33 * (128 * 32 * K * 2)
RP = EP // 128
RT_MID = RP // 16
RT_SPLIT = RP // 32
CH = 6272
EPS = 1e-5

RB = 2000
GRID = N // RB


def _make_sc_conv(two_tables: bool):
    nblocks = RT_MID // K if two_tables else RT_SPLIT // K
    npairs = nblocks // 2
    mesh = plsc.VectorSubcoreMesh(core_axis_name="c", subcore_axis_name="s")

    @functools.partial(
        pl.kernel,
        out_type=jax.ShapeDtypeStruct((2, N, 16), jnp.float32),
        mesh=mesh,
        scratch_types=[
            pltpu.VMEM((K, 2, 128), jnp.int32),
            pltpu.VMEM((K, 2, 128), jnp.int32),
            pltpu.VMEM((K, 128), jnp.float32),
            pltpu.VMEM((K, 128), jnp.float32),
            pltpu.VMEM((K, 128), jnp.int32),
            pltpu.VMEM((K, 128), jnp.int32),
            pltpu.VMEM((K * 128, 16), jnp.float32),
            pltpu.VMEM((K * 128, 16), jnp.float32),
            pltpu.VMEM_SHARED((N, 16), jnp.float32),
            pltpu.SemaphoreType.DMA,
            pltpu.SemaphoreType.DMA,
            pltpu.SemaphoreType.DMA,
            pltpu.SemaphoreType.DMA,
            pltpu.SemaphoreType.DMA,
            pltpu.SemaphoreType.DMA,
        ],
        compiler_params=pltpu.CompilerParams(use_tc_tiling_on_sc=False),
    )
    def conv(table, edges, wvals, zeros, out, idx0, idx1, w0, w1, dst0, dst1,
             rows0, rows1, acc, sem_i0, sem_i1, sem_g0, sem_g1, sem_s0,
             sem_s1):
        c = lax.axis_index("c")
        s = lax.axis_index("s")
        idx_b = (idx0, idx1)
        w_b = (w0, w1)
        dst_b = (dst0, dst1)
        rows_b = (rows0, rows1)
        sem_i = (sem_i0, sem_i1)
        sem_g = (sem_g0, sem_g1)
        sem_s = (sem_s0, sem_s1)
        n0 = jnp.minimum(s * CH, N - CH)
        pltpu.sync_copy(zeros.at[pl.ds(n0, CH)], acc.at[pl.ds(n0, CH)])
        plsc.subcore_barrier()
        if two_tables:
            row0 = s * RT_MID
            coff = c * N
        else:
            row0 = (c * 16 + s) * RT_SPLIT
            coff = None

        def fetch_idx(b, p):
            r = jnp.minimum(row0 + b * K, RP - K)
            d1 = pltpu.async_copy(edges.at[pl.ds(r, K)], idx_b[p], sem_i[p])
            d2 = pltpu.async_copy(wvals.at[pl.ds(r, K)], w_b[p], sem_i[p])
            return (d1, d2)

        def add_coff(p):
            if coff is None:
                return
            iv = idx_b[p]
            def body(q, car):
                sl = pl.ds((q % 8) * 16, 16)
                iv[q // 8, 0, sl] = iv[q // 8, 0, sl] + coff
                return car
            lax.fori_loop(0, K * 8, body, 0, unroll=2)

        def fire_gathers(p):
            return [
                pltpu.async_copy(table.at[idx_b[p].at[j, 0]],
                                 rows_b[p].at[pl.ds(j * 128, 128)], sem_g[p])
                for j in range(K)
            ]

        def mul_and_dst(p):
            iv, wv, dv, rv = idx_b[p], w_b[p], dst_b[p], rows_b[p]
            def body(q, car):
                j = q // 8
                sl = pl.ds((q % 8) * 16, 16)
                wc = wv[j, sl]
                dv[j, sl] = iv[j, 1, sl]
                base = q * 16
                for l in range(16):
                    rv[base + l, :] = rv[base + l, :] * wc[l]
                return car
            lax.fori_loop(0, K * 8, body, 0, unroll=2)

        def fire_scatters(p):
            return [
                pltpu.async_copy(rows_b[p].at[pl.ds(j * 128, 128)],
                                 acc.at[dst_b[p].at[j]], sem_s[p], add=True)
                for j in range(K)
            ]

        for d in fetch_idx(0, 0):
            d.wait()
        for d in fetch_idx(1, 1):
            d.wait()
        add_coff(0)
        add_coff(1)
        fire_gathers(0)
        fire_gathers(1)

        def pair_body(t, carry):
            for p in range(2):
                for j in range(K):
                    pltpu.make_async_copy(
                        table.at[idx_b[p].at[j, 0]],
                        rows_b[p].at[pl.ds(j * 128, 128)], sem_g[p]).wait()
                mul_and_dst(p)
                fire_scatters(p)
                dis = fetch_idx(2 * t + 2 + p, p)
                for j in range(K):
                    pltpu.make_async_copy(
                        rows_b[p].at[pl.ds(j * 128, 128)],
                        acc.at[dst_b[p].at[j]], sem_s[p]).wait()
                for d in dis:
                    d.wait()
                add_coff(p)
                fire_gathers(p)
            return carry

        lax.fori_loop(0, npairs, pair_body, 0)
        for p in range(2):
            for j in range(K):
                pltpu.make_async_copy(
                    table.at[idx_b[p].at[j, 0]],
                    rows_b[p].at[pl.ds(j * 128, 128)], sem_g[p]).wait()
        plsc.subcore_barrier()
        pltpu.sync_copy(acc.at[pl.ds(n0, CH)],
                        out.at[c, pl.ds(n0, CH)])

    return conv


_sc_mid = _make_sc_conv(True)
_sc_split = _make_sc_conv(False)


def _stats2(v):
    mu_b = jnp.mean(v, axis=0)
    d = v - mu_b
    return jnp.stack([mu_b, jnp.sum(d * d, axis=0)])[None]


def _combine(bs):
    mu = jnp.mean(bs[:, 0, :], axis=0)
    dmu = bs[:, 0, :] - mu
    var = (jnp.sum(bs[:, 1, :], axis=0) + RB * jnp.sum(dmu * dmu, axis=0)) / N
    return mu, lax.rsqrt(var + EPS)


def _bn0_stats_body(x_ref, st_ref):
    st_ref[...] = _stats2(x_ref[...])


def _zfirst_body(x_ref, st_ref, g_ref, b_ref, w_ref, bb_ref, z_ref):
    mu, rstd = _combine(st_ref[...])
    xb = (x_ref[...] - mu) * rstd * g_ref[...] + b_ref[...]
    z = jnp.dot(xb, w_ref[...], preferred_element_type=jnp.float32)
    z = z + bb_ref[...]
    z_ref[0] = z[:, 0:16]
    z_ref[1] = z[:, 16:32]


def _zfirst(x, g, b, w, bb):
    st = pl.pallas_call(
        _bn0_stats_body,
        grid=(GRID,),
        in_specs=[pl.BlockSpec((RB, 2), lambda i: (i, 0))],
        out_specs=pl.BlockSpec((1, 2, 2), lambda i: (i, 0, 0)),
        out_shape=jax.ShapeDtypeStruct((GRID, 2, 2), jnp.float32),
    )(x)
    return pl.pallas_call(
        _zfirst_body,
        grid=(GRID,),
        in_specs=[
            pl.BlockSpec((RB, 2), lambda i: (i, 0)),
            pl.BlockSpec((GRID, 2, 2), lambda i: (0, 0, 0)),
            pl.BlockSpec((1, 2), lambda i: (0, 0)),
            pl.BlockSpec((1, 2), lambda i: (0, 0)),
            pl.BlockSpec((2, H), lambda i: (0, 0)),
            pl.BlockSpec((1, H), lambda i: (0, 0)),
        ],
        out_specs=pl.BlockSpec((2, RB, 16), lambda i: (0, i, 0)),
        out_shape=jax.ShapeDtypeStruct((2, N, 16), jnp.float32),
    )(x, st, g, b, w, bb)


def _mid_stats_body(s_ref, st_ref):
    sv = jnp.concatenate([s_ref[0], s_ref[1]], axis=1)
    st_ref[...] = _stats2(sv)


def _mid_stats(s):
    return pl.pallas_call(
        _mid_stats_body,
        grid=(GRID,),
        in_specs=[pl.BlockSpec((2, RB, 16), lambda i: (0, i, 0))],
        out_specs=pl.BlockSpec((1, 2, H), lambda i: (i, 0, 0)),
        out_shape=jax.ShapeDtypeStruct((GRID, 2, H), jnp.float32),
    )(s)


def _mid_z_body(s_ref, st_ref, g_ref, b_ref, w_ref, bb_ref, z_ref):
    mu, rstd = _combine(st_ref[...])
    sv = jnp.concatenate([s_ref[0], s_ref[1]], axis=1)
    h = (sv - mu) * rstd * g_ref[...] + b_ref[...]
    h = jnp.maximum(h, 0.0)
    z = jnp.dot(h, w_ref[...], preferred_element_type=jnp.float32)
    z = z + bb_ref[...]
    z_ref[0] = z[:, 0:16]
    z_ref[1] = z[:, 16:32]


def _mid_z(s, st, g, b, w, bb):
    return pl.pallas_call(
        _mid_z_body,
        grid=(GRID,),
        in_specs=[
            pl.BlockSpec((2, RB, 16), lambda i: (0, i, 0)),
            pl.BlockSpec((GRID, 2, H), lambda i: (0, 0, 0)),
            pl.BlockSpec((1, H), lambda i: (0, 0)),
            pl.BlockSpec((1, H), lambda i: (0, 0)),
            pl.BlockSpec((H, H), lambda i: (0, 0)),
            pl.BlockSpec((1, H), lambda i: (0, 0)),
        ],
        out_specs=pl.BlockSpec((2, RB, 16), lambda i: (0, i, 0)),
        out_shape=jax.ShapeDtypeStruct((2, N, 16), jnp.float32),
    )(s, st, g, b, w, bb)


def _zlast_body(s_ref, st_ref, g_ref, b_ref, w_ref, bb_ref, z_ref):
    mu, rstd = _combine(st_ref[...])
    sv = jnp.concatenate([s_ref[0], s_ref[1]], axis=1)
    h = (sv - mu) * rstd * g_ref[...] + b_ref[...]
    h = jnp.maximum(h, 0.0)
    z = jnp.dot(h, w_ref[...], preferred_element_type=jnp.float32)
    z_ref[...] = z + bb_ref[...]


def _zlast(s, st, g, b, w, bb):
    return pl.pallas_call(
        _zlast_body,
        grid=(GRID,),
        in_specs=[
            pl.BlockSpec((2, RB, 16), lambda i: (0, i, 0)),
            pl.BlockSpec((GRID, 2, H), lambda i: (0, 0, 0)),
            pl.BlockSpec((1, H), lambda i: (0, 0)),
            pl.BlockSpec((1, H), lambda i: (0, 0)),
            pl.BlockSpec((H, OUT), lambda i: (0, 0)),
            pl.BlockSpec((1, OUT), lambda i: (0, 0)),
        ],
        out_specs=pl.BlockSpec((RB, OUT), lambda i: (i, 0)),
        out_shape=jax.ShapeDtypeStruct((N, OUT), jnp.float32),
    )(s, st, g, b, w, bb)


def _fin_stats_body(s_ref, st_ref):
    st_ref[...] = _stats2(s_ref[0] + s_ref[1])


def _fin_apply_body(s_ref, st_ref, g_ref, b_ref, o_ref):
    mu, rstd = _combine(st_ref[...])
    o = s_ref[0] + s_ref[1]
    o_ref[...] = (o - mu) * rstd * g_ref[...] + b_ref[...]


def _final(s, g, b):
    st = pl.pallas_call(
        _fin_stats_body,
        grid=(GRID,),
        in_specs=[pl.BlockSpec((2, RB, 16), lambda i: (0, i, 0))],
        out_specs=pl.BlockSpec((1, 2, OUT), lambda i: (i, 0, 0)),
        out_shape=jax.ShapeDtypeStruct((GRID, 2, OUT), jnp.float32),
    )(s)
    return pl.pallas_call(
        _fin_apply_body,
        grid=(GRID,),
        in_specs=[
            pl.BlockSpec((2, RB, 16), lambda i: (0, i, 0)),
            pl.BlockSpec((GRID, 2, OUT), lambda i: (0, 0, 0)),
            pl.BlockSpec((1, OUT), lambda i: (0, 0)),
            pl.BlockSpec((1, OUT), lambda i: (0, 0)),
        ],
        out_specs=pl.BlockSpec((RB, OUT), lambda i: (i, 0)),
        out_shape=jax.ShapeDtypeStruct((N, OUT), jnp.float32),
    )(s, st, g, b)


def kernel(x, edge_index, edge_weight, bn0_gamma, bn0_beta, W_first, b_first,
           bn1_gamma, bn1_beta, W_mid, b_mid, bnm_gamma, bnm_beta,
           W_last, b_last, bnl_gamma, bnl_beta):
    pad = EP - E
    src = jnp.concatenate([edge_index[0], jnp.zeros((pad,), jnp.int32)])
    dst = jnp.concatenate([edge_index[1], jnp.zeros((pad,), jnp.int32)])
    wgt = jnp.concatenate([edge_weight, jnp.zeros((pad,), jnp.float32)])
    edges = jnp.stack([src.reshape(RP, 128), dst.reshape(RP, 128)], axis=1)
    wvals = wgt.reshape(RP, 128)
    zeros16 = jnp.zeros((N, 16), jnp.float32)

    z = _zfirst(x, bn0_gamma.reshape(1, 2), bn0_beta.reshape(1, 2),
                W_first, b_first.reshape(1, H))
    s = _sc_mid(z.reshape(2 * N, 16), edges, wvals, zeros16)
    st = _mid_stats(s)
    g, b = bn1_gamma, bn1_beta
    for i in range(NMID):
        z = _mid_z(s, st, g.reshape(1, H), b.reshape(1, H),
                   W_mid[i], b_mid[i].reshape(1, H))
        s = _sc_mid(z.reshape(2 * N, 16), edges, wvals, zeros16)
        st = _mid_stats(s)
        g, b = bnm_gamma[i], bnm_beta[i]
    zl = _zlast(s, st, g.reshape(1, H), b.reshape(1, H),
                W_last, b_last.reshape(1, OUT))
    sl = _sc_split(zl, edges, wvals, zeros16)
    return _final(sl, bnl_gamma.reshape(1, OUT), bnl_beta.reshape(1, OUT))
